# baseline (speedup 1.0000x reference)
import jax
import jax.numpy as jnp
from jax import lax
from jax.experimental import pallas as pl
from jax.experimental.pallas import tpu as pltpu
from jax.experimental.pallas import tpu_sc as plsc

NC = 2
NS = 16
NW = NC * NS
G = 128
D = 128
B = 400000
R = (B + G - 1) // G
NCH = 104
PR = NW * NCH
NBUF = 4
LOOK = 2


def _gather_body(table_hbm, idx_hbm, out_hbm, idx_v, rows_v, *sems):
    gsems = sems[:NBUF]
    wsems = sems[NBUF:]
    wid = lax.axis_index("s") * NC + lax.axis_index("c")
    base = wid * NCH

    pltpu.sync_copy(idx_hbm.at[pl.ds(base, NCH)], idx_v)

    def gstart(j, b):
        pltpu.async_copy(table_hbm.at[idx_v.at[j]], rows_v.at[b], gsems[b])

    def gwait(b):
        pltpu.make_async_copy(
            table_hbm.at[pl.ds(0, G)], rows_v.at[b], gsems[b]).wait()

    def wstart(j, b):
        pltpu.async_copy(
            rows_v.at[b], out_hbm.at[pl.ds((base + j) * G, G)], wsems[b])

    def wwait(b):
        pltpu.make_async_copy(
            rows_v.at[b], out_hbm.at[pl.ds(0, G)], wsems[b]).wait()

    gstart(0, 0)
    gstart(1, 1)

    def outer(g, carry):
        j0 = g * NBUF
        for b in range(NBUF):
            j = j0 + b
            jn = j + LOOK
            bn = (b + LOOK) % NBUF
            @pl.when((jn < NCH) & (j >= LOOK) & (base + j - LOOK < R))
            def _():
                wwait(bn)

            @pl.when(jn < NCH)
            def _():
                gstart(jn, bn)

            gwait(b)

            @pl.when(base + j < R)
            def _():
                wstart(j, b)

        return carry

    lax.fori_loop(0, NCH // NBUF, outer, 0)

    @pl.when(base + NCH - 2 < R)
    def _():
        wwait((NCH - 2) % NBUF)

    @pl.when(base + NCH - 1 < R)
    def _():
        wwait((NCH - 1) % NBUF)


def kernel(x, upsample):
    idx = upsample[:, 0].astype(jnp.int32)
    idx = jnp.concatenate(
        [idx, jnp.zeros((PR * G - B,), jnp.int32)]).reshape(PR, G)
    table = jnp.concatenate([x, jnp.zeros((1, D), x.dtype)], axis=0)
    f = pl.kernel(
        _gather_body,
        out_type=jax.ShapeDtypeStruct((B, D), jnp.float32),
        mesh=plsc.VectorSubcoreMesh(core_axis_name="c", subcore_axis_name="s"),
        scratch_types=(
            [pltpu.VMEM((NCH, G), jnp.int32),
             pltpu.VMEM((NBUF, G, D), jnp.float32)]
            + [pltpu.SemaphoreType.DMA] * (2 * NBUF)
        ),
    )
    return f(table, idx)


# speedup vs baseline: 4.4503x; 4.4503x over previous
import jax
import jax.numpy as jnp
from jax import lax
from jax.experimental import pallas as pl
from jax.experimental.pallas import tpu as pltpu
from jax.experimental.pallas import tpu_sc as plsc

NC = 2
NS = 16
NW = NC * NS
G = 128
D = 128
B = 400000
R = B // G
NIT = (R + NW - 1) // NW


def _gather_body(table_hbm, idx_hbm, out_hbm, idx_r, rows_r, gsem, ws0, ws1):
    wsems = (ws0, ws1)
    wid = lax.axis_index("s") * NC + lax.axis_index("c")

    def wwait(b):
        pltpu.make_async_copy(
            rows_r.at[b], out_hbm.at[pl.ds(0, G)], wsems[b]).wait()

    def step(g, carry):
        for b in range(2):
            i = 2 * g + b
            row = wid + i * NW

            @pl.when(row < R)
            def _():
                @pl.when(i >= 2)
                def _():
                    wwait(b)

                pltpu.sync_copy(idx_hbm.at[row], idx_r.at[b])
                pltpu.async_copy(
                    table_hbm.at[idx_r.at[b]], rows_r.at[b], gsem).wait()
                pltpu.async_copy(
                    rows_r.at[b], out_hbm.at[pl.ds(row * G, G)], wsems[b])

        return carry

    lax.fori_loop(0, NIT // 2, step, 0)

    wwait(0)
    wwait(1)


def kernel(x, upsample):
    idx = upsample[:, 0].astype(jnp.int32).reshape(R, G)
    table = jnp.concatenate([x, jnp.zeros((1, D), x.dtype)], axis=0)
    f = pl.kernel(
        _gather_body,
        out_type=jax.ShapeDtypeStruct((B, D), jnp.float32),
        mesh=plsc.VectorSubcoreMesh(core_axis_name="c", subcore_axis_name="s"),
        scratch_types=[
            pltpu.VMEM((2, G), jnp.int32),
            pltpu.VMEM((2, G, D), jnp.float32),
            pltpu.SemaphoreType.DMA,
            pltpu.SemaphoreType.DMA,
            pltpu.SemaphoreType.DMA,
        ],
    )
    return f(table, idx)


# speedup vs baseline: 6.0342x; 1.3559x over previous
import jax
import jax.numpy as jnp
from jax import lax
from jax.experimental import pallas as pl
from jax.experimental.pallas import tpu as pltpu
from jax.experimental.pallas import tpu_sc as plsc

NC = 2
NS = 16
NW = NC * NS
G = 128
D = 128
B = 400000
R = B // G
NIT = (R + NW - 1) // NW
NBUF = 4
STEPS = 100


def _gather_body(table_hbm, idx_hbm, out_hbm, idx_r, rows_r, *sems):
    isems = sems[:NBUF]
    gsems = sems[NBUF:2 * NBUF]
    wsems = sems[2 * NBUF:]
    wid = lax.axis_index("s") * NC + lax.axis_index("c")

    def valid(i):
        return wid + i * NW < R

    def istart(i, b):
        pltpu.async_copy(idx_hbm.at[wid + i * NW], idx_r.at[b], isems[b])

    def iwait(b):
        pltpu.make_async_copy(
            idx_hbm.at[0], idx_r.at[b], isems[b]).wait()

    def gstart(i, b):
        pltpu.async_copy(table_hbm.at[idx_r.at[b]], rows_r.at[b], gsems[b])

    def gwait(b):
        pltpu.make_async_copy(
            table_hbm.at[pl.ds(0, G)], rows_r.at[b], gsems[b]).wait()

    def wstart(i, b):
        pltpu.async_copy(
            rows_r.at[b], out_hbm.at[pl.ds((wid + i * NW) * G, G)], wsems[b])

    def wwait(b):
        pltpu.make_async_copy(
            rows_r.at[b], out_hbm.at[pl.ds(0, G)], wsems[b]).wait()

    @pl.when(valid(0))
    def _():
        istart(0, 0)
        iwait(0)
        gstart(0, 0)

    @pl.when(valid(1))
    def _():
        istart(1, 1)

    def step(g, carry):
        for b in range(NBUF):
            i4 = g * NBUF
            bn = (b + 1) % NBUF
            bp = (b + 2) % NBUF

            @pl.when(valid(i4 + b - 3) & (i4 + b >= 3))
            def _():
                wwait(bn)

            @pl.when(valid(i4 + b + 2))
            def _():
                istart(i4 + b + 2, bp)

            @pl.when(valid(i4 + b + 1))
            def _():
                iwait(bn)
                gstart(i4 + b + 1, bn)

            @pl.when(valid(i4 + b))
            def _():
                gwait(b)
                wstart(i4 + b, b)

        return carry

    lax.fori_loop(0, STEPS // NBUF, step, 0)

    @pl.when(valid(NIT - 1))
    def _():
        wwait((NIT - 1) % NBUF)


def kernel(x, upsample):
    idx = upsample[:, 0].astype(jnp.int32).reshape(R, G)
    table = jnp.concatenate([x, jnp.zeros((1, D), x.dtype)], axis=0)
    f = pl.kernel(
        _gather_body,
        out_type=jax.ShapeDtypeStruct((B, D), jnp.float32),
        mesh=plsc.VectorSubcoreMesh(core_axis_name="c", subcore_axis_name="s"),
        scratch_types=(
            [pltpu.VMEM((NBUF, G), jnp.int32),
             pltpu.VMEM((NBUF, G, D), jnp.float32)]
            + [pltpu.SemaphoreType.DMA] * (3 * NBUF)
        ),
    )
    return f(table, idx)
